# Initial kernel scaffold; baseline (speedup 1.0000x reference)
import jax
import jax.numpy as jnp
from jax.experimental import pallas as pl


def kernel(x, edge_index, W1, b1, W2, b2):
    raise NotImplementedError("write your pallas kernel here")


# speedup vs baseline: 4.4520x; 4.4520x over previous
import functools

import jax
import jax.numpy as jnp
from jax import lax
from jax.experimental import pallas as pl
from jax.experimental.pallas import tpu as pltpu
from jax.experimental.pallas import tpu_sc as plsc

N = 10000
E = 320000
D = 128
K = 3

NW = 32
CH = 128
NCHUNK = 80
EPT = NCHUNK * CH
EP = NW * EPT
NP = 10112
RPT = NP // 16
R = 1000
G = N // R

_mesh = plsc.VectorSubcoreMesh(core_axis_name="c", subcore_axis_name="s")


def _sc_degree_body(colc_hbm, ones_hbm, z_hbm, out_hbm, colv, onesv, acc):
    cid = lax.axis_index("c")
    sid = lax.axis_index("s")
    wid = sid * 2 + cid
    pltpu.sync_copy(colc_hbm.at[wid], colv)
    pltpu.sync_copy(ones_hbm, onesv)
    pltpu.sync_copy(z_hbm, acc.at[pl.ds(sid * RPT, RPT)])
    plsc.subcore_barrier()

    def body(j, carry):
        pltpu.sync_copy(onesv, acc.at[colv.at[j]], add=True)
        return carry

    lax.fori_loop(0, NCHUNK, body, 0)
    plsc.subcore_barrier()
    pltpu.sync_copy(acc.at[pl.ds(sid * RPT, RPT)],
                    out_hbm.at[cid, pl.ds(sid * RPT, RPT)])


def _sc_scatter_body(g_hbm, rowc_hbm, colc_hbm, z_hbm, out_hbm,
                     rowv, colv, bufa, bufb, acc, sema, semb):
    cid = lax.axis_index("c")
    sid = lax.axis_index("s")
    wid = sid * 2 + cid
    half = NCHUNK // 2
    pltpu.sync_copy(z_hbm, acc.at[pl.ds(sid * RPT, RPT)])
    plsc.subcore_barrier()

    for phase in range(2):
        pltpu.sync_copy(rowc_hbm.at[wid, pl.ds(phase * half, half)], rowv)
        pltpu.sync_copy(colc_hbm.at[wid, pl.ds(phase * half, half)], colv)
        pltpu.async_copy(g_hbm.at[rowv.at[0]], bufa, sema)

        def pair(i, carry):
            j = 2 * i
            pltpu.async_copy(g_hbm.at[rowv.at[j + 1]], bufb, semb)
            pltpu.make_async_copy(g_hbm.at[rowv.at[j]], bufa, sema).wait()
            pltpu.sync_copy(bufa, acc.at[colv.at[j]], add=True)

            @pl.when(j + 2 < half)
            def _():
                pltpu.async_copy(g_hbm.at[rowv.at[j + 2]], bufa, sema)

            pltpu.make_async_copy(g_hbm.at[rowv.at[j + 1]], bufb, semb).wait()
            pltpu.sync_copy(bufb, acc.at[colv.at[j + 1]], add=True)
            return carry

        lax.fori_loop(0, half // 2, pair, 0)
    plsc.subcore_barrier()
    pltpu.sync_copy(acc.at[pl.ds(sid * RPT, RPT)],
                    out_hbm.at[cid, pl.ds(sid * RPT, RPT)])


def _make_sc_kernels(interpret=False):
    deg = pl.kernel(
        _sc_degree_body,
        out_type=jax.ShapeDtypeStruct((2, NP, D), jnp.float32),
        mesh=_mesh,
        scratch_types=[
            pltpu.VMEM((NCHUNK, CH), jnp.int32),
            pltpu.VMEM((CH, D), jnp.float32),
            pltpu.VMEM_SHARED((NP, D), jnp.float32),
        ],
        interpret=interpret,
    )
    scat = pl.kernel(
        _sc_scatter_body,
        out_type=jax.ShapeDtypeStruct((2, NP, D), jnp.float32),
        mesh=_mesh,
        scratch_types=[
            pltpu.VMEM((NCHUNK // 2, CH), jnp.int32),
            pltpu.VMEM((NCHUNK // 2, CH), jnp.int32),
            pltpu.VMEM((CH, D), jnp.float32),
            pltpu.VMEM((CH, D), jnp.float32),
            pltpu.VMEM_SHARED((NP, D), jnp.float32),
            pltpu.SemaphoreType.DMA,
            pltpu.SemaphoreType.DMA,
        ],
        interpret=interpret,
    )
    return deg, scat


_sc_degree, _sc_scatter = _make_sc_kernels()


def _prep_body(degp0_ref, degp1_ref, x_ref, w_ref, dinv_ref, g_ref, out_ref):
    d = degp0_ref[0] + degp1_ref[0]
    d0 = d[:, 0:1]
    dinv = jnp.where(d0 > 0, 1.0 / jnp.sqrt(d0), 0.0)
    dinvb = jnp.broadcast_to(dinv, (R, D))
    x = x_ref[...]
    dinv_ref[...] = dinvb
    g_ref[...] = dinvb * x
    out_ref[...] = jnp.dot(x, w_ref[...], preferred_element_type=jnp.float32)


def _mid_body(p_ref0, p_ref1, dinv_ref, w_ref, oin_ref, oout_ref, g_ref):
    dinv = dinv_ref[...]
    h = dinv * (p_ref0[0] + p_ref1[0])
    oout_ref[...] = oin_ref[...] + jnp.dot(h, w_ref[...],
                                           preferred_element_type=jnp.float32)
    g_ref[...] = dinv * h


def _end1_body(p_ref0, p_ref1, dinv_ref, w_ref, b_ref, w20_ref, oin_ref,
               oout_ref, g_ref):
    dinv = dinv_ref[...]
    h = dinv * (p_ref0[0] + p_ref1[0])
    o = oin_ref[...] + jnp.dot(h, w_ref[...],
                               preferred_element_type=jnp.float32) + b_ref[...]
    a = jnp.maximum(o, 0.0)
    g_ref[...] = dinv * a
    oout_ref[...] = jnp.dot(a, w20_ref[...], preferred_element_type=jnp.float32)


def _end2_body(p_ref0, p_ref1, dinv_ref, w_ref, b_ref, oin_ref, out_ref):
    dinv = dinv_ref[...]
    h = dinv * (p_ref0[0] + p_ref1[0])
    z = oin_ref[...] + jnp.dot(h, w_ref[...],
                               preferred_element_type=jnp.float32) + b_ref[...]
    m = jnp.max(z, axis=1, keepdims=True)
    zs = z - m
    out_ref[...] = zs - jnp.log(jnp.sum(jnp.exp(zs), axis=1, keepdims=True))


_spec_r = pl.BlockSpec((R, D), lambda i: (i, 0))
_spec_p0 = pl.BlockSpec((1, R, D), lambda i: (0, i, 0))
_spec_p1 = pl.BlockSpec((1, R, D), lambda i: (1, i, 0))
_spec_w = pl.BlockSpec((D, D), lambda i: (0, 0))
_spec_b = pl.BlockSpec((1, D), lambda i: (0, 0))
_out_r = jax.ShapeDtypeStruct((N, D), jnp.float32)

def _make_tc_kernels(interpret=False):
    prep = pl.pallas_call(
        _prep_body,
        grid=(G,),
        in_specs=[_spec_p0, _spec_p1, _spec_r, _spec_w],
        out_specs=[_spec_r, _spec_r, _spec_r],
        out_shape=[_out_r, _out_r, _out_r],
        interpret=interpret,
    )
    mid = pl.pallas_call(
        _mid_body,
        grid=(G,),
        in_specs=[_spec_p0, _spec_p1, _spec_r, _spec_w, _spec_r],
        out_specs=[_spec_r, _spec_r],
        out_shape=[_out_r, _out_r],
        interpret=interpret,
    )
    end1 = pl.pallas_call(
        _end1_body,
        grid=(G,),
        in_specs=[_spec_p0, _spec_p1, _spec_r, _spec_w, _spec_b, _spec_w,
                  _spec_r],
        out_specs=[_spec_r, _spec_r],
        out_shape=[_out_r, _out_r],
        interpret=interpret,
    )
    end2 = pl.pallas_call(
        _end2_body,
        grid=(G,),
        in_specs=[_spec_p0, _spec_p1, _spec_r, _spec_w, _spec_b, _spec_r],
        out_specs=_spec_r,
        out_shape=_out_r,
        interpret=interpret,
    )
    return prep, mid, end1, end2


_tc_prep, _tc_mid, _tc_end1, _tc_end2 = _make_tc_kernels()


def kernel(x, edge_index, W1, b1, W2, b2):
    row = edge_index[0]
    col = edge_index[1]
    pad = EP - E
    rowc = jnp.concatenate(
        [row, jnp.zeros((pad,), jnp.int32)]).reshape(NW, NCHUNK, CH)
    colc = jnp.concatenate(
        [col, jnp.full((pad,), N, jnp.int32)]).reshape(NW, NCHUNK, CH)
    zrows = jnp.zeros((RPT, D), jnp.float32)
    onesc = jnp.ones((CH, D), jnp.float32)

    degp = _sc_degree(colc, onesc, zrows)
    dinvb, g, oacc = _tc_prep(degp, degp, x, W1[0])

    b1r = b1.reshape(1, D)
    b2r = b2.reshape(1, D)

    for k in (1, 2):
        p = _sc_scatter(g, rowc, colc, zrows)
        oacc, g = _tc_mid(p, p, dinvb, W1[k], oacc)
    p = _sc_scatter(g, rowc, colc, zrows)
    oacc, g = _tc_end1(p, p, dinvb, W1[3], b1r, W2[0], oacc)

    for k in (1, 2):
        p = _sc_scatter(g, rowc, colc, zrows)
        oacc, g = _tc_mid(p, p, dinvb, W2[k], oacc)
    p = _sc_scatter(g, rowc, colc, zrows)
    return _tc_end2(p, p, dinvb, W2[3], b2r, oacc)
